# Initial kernel scaffold; baseline (speedup 1.0000x reference)
import jax
import jax.numpy as jnp
from jax.experimental import pallas as pl


def kernel(x, edge_index, batch, W1, b1, W2, b2, Wlin, blin):
    raise NotImplementedError("write your pallas kernel here")


# speedup vs baseline: 11.7296x; 11.7296x over previous
import functools

import jax
import jax.numpy as jnp
from jax import lax
from jax.experimental import pallas as pl
from jax.experimental.pallas import tpu as pltpu
from jax.experimental.pallas import tpu_sc as plsc

_N = 10000
_NP = 10240
_D = 128
_G = 64
_NC = 2
_NS = 16
_NW = _NC * _NS
_K = 80
_RB = 512


def _sc_aggregate(u, src, dst):
    e = src.shape[0]
    ew = e // _NW
    nch = ew // _K
    rpt = _NP // _NS
    mesh = plsc.VectorSubcoreMesh(core_axis_name="c", subcore_axis_name="s")

    @functools.partial(
        pl.kernel,
        out_type=jax.ShapeDtypeStruct((_NC, _NP, _D), jnp.float32),
        mesh=mesh,
        scratch_types=[
            pltpu.VMEM_SHARED((_NP, _D), jnp.float32),
            pltpu.VMEM((ew,), jnp.int32),
            pltpu.VMEM((_K,), jnp.int32),
            pltpu.VMEM((_K, _D), jnp.float32),
            pltpu.VMEM((128, _D), jnp.float32),
            pltpu.SemaphoreType.DMA,
        ],
    )
    def agg_kernel(u_hbm, src_hbm, dst_hbm, out_hbm,
                   acc, sidx, didx80, rows, zb, sem):
        c = lax.axis_index("c")
        s = lax.axis_index("s")
        wid = c * _NS + s
        base = wid * ew
        z16 = jnp.zeros((16,), jnp.float32)

        def _fill(i, _):
            for j in range(_D // 16):
                zb[i, pl.ds(j * 16, 16)] = z16
            return 0

        lax.fori_loop(0, 128, _fill, 0)

        pltpu.sync_copy(src_hbm.at[pl.ds(base, ew)], sidx)

        def _zcp(j, _):
            pltpu.sync_copy(zb, acc.at[pl.ds(s * rpt + j * 128, 128)])
            return 0

        lax.fori_loop(0, rpt // 128, _zcp, 0)
        plsc.subcore_barrier()

        def _chunk(k, _):
            pltpu.sync_copy(dst_hbm.at[pl.ds(base + k * _K, _K)], didx80)
            pltpu.async_copy(u_hbm.at[sidx.at[pl.ds(k * _K, _K)]],
                             rows, sem).wait()
            pltpu.sync_copy(rows, acc.at[didx80], add=True)
            return 0

        lax.fori_loop(0, nch, _chunk, 0)
        plsc.subcore_barrier()
        pltpu.sync_copy(acc.at[pl.ds(s * rpt, rpt)],
                        out_hbm.at[c, pl.ds(s * rpt, rpt)])

    return agg_kernel(u, src, dst)


def _tc_pre(xp, w1, degp):
    nb = _NP // _RB

    def body(x_ref, w_ref, d0_ref, d1_ref, u_ref, dv_ref):
        deg = d0_ref[0, :, 0:1] + d1_ref[0, :, 0:1] + 1.0
        dv = lax.rsqrt(deg)
        dv_ref[...] = dv
        u_ref[...] = jnp.dot(x_ref[...], w_ref[...],
                             preferred_element_type=jnp.float32) * dv

    return pl.pallas_call(
        body,
        grid=(nb,),
        in_specs=[
            pl.BlockSpec((_RB, _D), lambda i: (i, 0)),
            pl.BlockSpec((_D, _D), lambda i: (0, 0)),
            pl.BlockSpec((1, _RB, _D), lambda i: (0, i, 0)),
            pl.BlockSpec((1, _RB, _D), lambda i: (1, i, 0)),
        ],
        out_specs=[
            pl.BlockSpec((_RB, _D), lambda i: (i, 0)),
            pl.BlockSpec((_RB, 1), lambda i: (i, 0)),
        ],
        out_shape=[
            jax.ShapeDtypeStruct((_NP, _D), jnp.float32),
            jax.ShapeDtypeStruct((_NP, 1), jnp.float32),
        ],
    )(xp, w1, degp, degp)


def _tc_mid(u1, p, dinv, b1, w2):
    nb = _NP // _RB

    def body(u_ref, p0_ref, p1_ref, dv_ref, b_ref, w_ref, o_ref):
        dv = dv_ref[...]
        s = (u_ref[...] + p0_ref[0] + p1_ref[0]) * dv + b_ref[...]
        h = jnp.maximum(s, 0.0)
        o_ref[...] = jnp.dot(h, w_ref[...],
                             preferred_element_type=jnp.float32) * dv

    return pl.pallas_call(
        body,
        grid=(nb,),
        in_specs=[
            pl.BlockSpec((_RB, _D), lambda i: (i, 0)),
            pl.BlockSpec((1, _RB, _D), lambda i: (0, i, 0)),
            pl.BlockSpec((1, _RB, _D), lambda i: (1, i, 0)),
            pl.BlockSpec((_RB, 1), lambda i: (i, 0)),
            pl.BlockSpec((1, _D), lambda i: (0, 0)),
            pl.BlockSpec((_D, _D), lambda i: (0, 0)),
        ],
        out_specs=pl.BlockSpec((_RB, _D), lambda i: (i, 0)),
        out_shape=jax.ShapeDtypeStruct((_NP, _D), jnp.float32),
    )(u1, p, p, dinv, b1, w2)


def _tc_post(u2, q, dinv, b2, batch2d, wlin, blin):
    nb = _NP // _RB
    c_out = wlin.shape[1]

    def body(u_ref, q0_ref, q1_ref, dv_ref, b_ref, bt_ref, wl_ref, bl_ref,
             o_ref, pool_acc, cnt_acc):
        i = pl.program_id(0)
        dv = dv_ref[...]
        s = (u_ref[...] + q0_ref[0] + q1_ref[0]) * dv + b_ref[...]
        h = jnp.maximum(s, 0.0)
        oh = (bt_ref[...] == lax.broadcasted_iota(jnp.int32, (_RB, _G), 1))
        oh = oh.astype(jnp.float32)
        pp = lax.dot_general(oh, h, (((0,), (0,)), ((), ())),
                             preferred_element_type=jnp.float32)
        cp = lax.dot_general(oh, jnp.ones((_RB, 1), jnp.float32),
                             (((0,), (0,)), ((), ())),
                             preferred_element_type=jnp.float32)

        @pl.when(i == 0)
        def _():
            pool_acc[...] = pp
            cnt_acc[...] = cp

        @pl.when(i > 0)
        def _():
            pool_acc[...] += pp
            cnt_acc[...] += cp

        @pl.when(i == nb - 1)
        def _():
            pooled = pool_acc[...] / jnp.maximum(cnt_acc[...], 1.0)
            o_ref[...] = jnp.dot(pooled, wl_ref[...],
                                 preferred_element_type=jnp.float32) + bl_ref[...]

    return pl.pallas_call(
        body,
        grid=(nb,),
        in_specs=[
            pl.BlockSpec((_RB, _D), lambda i: (i, 0)),
            pl.BlockSpec((1, _RB, _D), lambda i: (0, i, 0)),
            pl.BlockSpec((1, _RB, _D), lambda i: (1, i, 0)),
            pl.BlockSpec((_RB, 1), lambda i: (i, 0)),
            pl.BlockSpec((1, _D), lambda i: (0, 0)),
            pl.BlockSpec((_RB, 1), lambda i: (i, 0)),
            pl.BlockSpec((_D, c_out), lambda i: (0, 0)),
            pl.BlockSpec((1, c_out), lambda i: (0, 0)),
        ],
        out_specs=pl.BlockSpec((_G, c_out), lambda i: (0, 0)),
        out_shape=jax.ShapeDtypeStruct((_G, c_out), jnp.float32),
        scratch_shapes=[
            pltpu.VMEM((_G, _D), jnp.float32),
            pltpu.VMEM((_G, 1), jnp.float32),
        ],
    )(u2, q, q, dinv, b2, batch2d, wlin, blin)


def kernel(x, edge_index, batch, W1, b1, W2, b2, Wlin, blin):
    src = edge_index[0]
    dst = edge_index[1]
    xp = jnp.pad(x, ((0, _NP - _N), (0, 0)))
    batchp = jnp.pad(batch, (0, _NP - _N), constant_values=_G)
    ones = jnp.ones((_NP, _D), jnp.float32)
    degp = _sc_aggregate(ones, dst, dst)
    u1, dinv = _tc_pre(xp, W1, degp)
    p = _sc_aggregate(u1, src, dst)
    u2 = _tc_mid(u1, p, dinv, b1.reshape(1, _D), W2)
    q = _sc_aggregate(u2, src, dst)
    return _tc_post(u2, q, dinv, b2.reshape(1, _D),
                    batchp.reshape(_NP, 1), Wlin, blin.reshape(1, -1))


# speedup vs baseline: 25.7895x; 2.1987x over previous
import functools

import jax
import jax.numpy as jnp
from jax import lax
from jax.experimental import pallas as pl
from jax.experimental.pallas import tpu as pltpu
from jax.experimental.pallas import tpu_sc as plsc

_N = 10000
_NP = 10240
_D = 128
_G = 64
_NC = 2
_NS = 16
_NW = _NC * _NS
_K = 80
_RB = 512


def _sc_count(dst):
    e = dst.shape[0]
    ew = e // _NW
    nch = ew // _K
    rpt = _NP // _NS
    mesh = plsc.VectorSubcoreMesh(core_axis_name="c", subcore_axis_name="s")

    @functools.partial(
        pl.kernel,
        out_type=jax.ShapeDtypeStruct((_NC, _NP, _D), jnp.float32),
        mesh=mesh,
        scratch_types=[
            pltpu.VMEM_SHARED((_NP, _D), jnp.float32),
            pltpu.VMEM((_K,), jnp.int32),
            pltpu.VMEM((_K,), jnp.int32),
            pltpu.VMEM((_K, _D), jnp.float32),
            pltpu.VMEM((128, _D), jnp.float32),
            pltpu.SemaphoreType.DMA,
            pltpu.SemaphoreType.DMA,
        ],
    )
    def cnt_kernel(dst_hbm, out_hbm, acc, di0, di1, ones_b, zb, dm0, dm1):
        c = lax.axis_index("c")
        s = lax.axis_index("s")
        wid = c * _NS + s
        base = wid * ew
        z16 = jnp.zeros((16,), jnp.float32)
        o16 = jnp.full((16,), 1.0, jnp.float32)

        def _fill(i, _):
            for j in range(_D // 16):
                zb[i, pl.ds(j * 16, 16)] = z16
            return 0

        lax.fori_loop(0, 128, _fill, 0)

        def _fill1(i, _):
            for j in range(_D // 16):
                ones_b[i, pl.ds(j * 16, 16)] = o16
            return 0

        lax.fori_loop(0, _K, _fill1, 0)

        def _zcp(j, _):
            pltpu.sync_copy(zb, acc.at[pl.ds(s * rpt + j * 128, 128)])
            return 0

        lax.fori_loop(0, rpt // 128, _zcp, 0)
        plsc.subcore_barrier()

        pltpu.async_copy(dst_hbm.at[pl.ds(base, _K)], di0, dm0)

        def _pair(k, _):
            c0 = 2 * k
            pltpu.async_copy(dst_hbm.at[pl.ds(base + (c0 + 1) * _K, _K)],
                             di1, dm1)
            pltpu.make_async_copy(dst_hbm.at[pl.ds(base + c0 * _K, _K)],
                                  di0, dm0).wait()
            pltpu.sync_copy(ones_b, acc.at[di0], add=True)
            pltpu.async_copy(dst_hbm.at[pl.ds(base + (c0 + 2) * _K, _K)],
                             di0, dm0)
            pltpu.make_async_copy(dst_hbm.at[pl.ds(base + (c0 + 1) * _K, _K)],
                                  di1, dm1).wait()
            pltpu.sync_copy(ones_b, acc.at[di1], add=True)
            return 0

        lax.fori_loop(0, (nch - 1) // 2, _pair, 0)

        pltpu.make_async_copy(dst_hbm.at[pl.ds(base + (nch - 1) * _K, _K)],
                              di0, dm0).wait()
        pltpu.sync_copy(ones_b, acc.at[di0], add=True)

        plsc.subcore_barrier()
        pltpu.sync_copy(acc.at[pl.ds(s * rpt, rpt)],
                        out_hbm.at[c, pl.ds(s * rpt, rpt)])

    return cnt_kernel(dst)


def _sc_aggregate(u, src, dst):
    e = src.shape[0]
    ew = e // _NW
    nch = ew // _K
    rpt = _NP // _NS
    mesh = plsc.VectorSubcoreMesh(core_axis_name="c", subcore_axis_name="s")

    @functools.partial(
        pl.kernel,
        out_type=jax.ShapeDtypeStruct((_NC, _NP, _D), jnp.float32),
        mesh=mesh,
        scratch_types=[
            pltpu.VMEM_SHARED((_NP, _D), jnp.float32),
            pltpu.VMEM((ew,), jnp.int32),
            pltpu.VMEM((_K,), jnp.int32),
            pltpu.VMEM((_K,), jnp.int32),
            pltpu.VMEM((_K, _D), jnp.float32),
            pltpu.VMEM((_K, _D), jnp.float32),
            pltpu.VMEM((128, _D), jnp.float32),
            pltpu.SemaphoreType.DMA,
            pltpu.SemaphoreType.DMA,
            pltpu.SemaphoreType.DMA,
            pltpu.SemaphoreType.DMA,
        ],
    )
    def agg_kernel(u_hbm, src_hbm, dst_hbm, out_hbm, acc, sidx,
                   di0, di1, rows0, rows1, zb, sem0, sem1, dm0, dm1):
        c = lax.axis_index("c")
        s = lax.axis_index("s")
        wid = c * _NS + s
        base = wid * ew
        z16 = jnp.zeros((16,), jnp.float32)

        def _fill(i, _):
            for j in range(_D // 16):
                zb[i, pl.ds(j * 16, 16)] = z16
            return 0

        lax.fori_loop(0, 128, _fill, 0)

        pltpu.sync_copy(src_hbm.at[pl.ds(base, ew)], sidx)

        def _zcp(j, _):
            pltpu.sync_copy(zb, acc.at[pl.ds(s * rpt + j * 128, 128)])
            return 0

        lax.fori_loop(0, rpt // 128, _zcp, 0)
        plsc.subcore_barrier()

        pltpu.async_copy(dst_hbm.at[pl.ds(base, _K)], di0, dm0)
        pltpu.async_copy(u_hbm.at[sidx.at[pl.ds(0, _K)]], rows0, sem0)

        def _pair(k, _):
            c0 = 2 * k
            pltpu.async_copy(dst_hbm.at[pl.ds(base + (c0 + 1) * _K, _K)],
                             di1, dm1)
            pltpu.async_copy(u_hbm.at[sidx.at[pl.ds((c0 + 1) * _K, _K)]],
                             rows1, sem1)
            pltpu.make_async_copy(dst_hbm.at[pl.ds(base + c0 * _K, _K)],
                                  di0, dm0).wait()
            pltpu.make_async_copy(u_hbm.at[sidx.at[pl.ds(c0 * _K, _K)]],
                                  rows0, sem0).wait()
            pltpu.sync_copy(rows0, acc.at[di0], add=True)
            pltpu.async_copy(dst_hbm.at[pl.ds(base + (c0 + 2) * _K, _K)],
                             di0, dm0)
            pltpu.async_copy(u_hbm.at[sidx.at[pl.ds((c0 + 2) * _K, _K)]],
                             rows0, sem0)
            pltpu.make_async_copy(dst_hbm.at[pl.ds(base + (c0 + 1) * _K, _K)],
                                  di1, dm1).wait()
            pltpu.make_async_copy(u_hbm.at[sidx.at[pl.ds((c0 + 1) * _K, _K)]],
                                  rows1, sem1).wait()
            pltpu.sync_copy(rows1, acc.at[di1], add=True)
            return 0

        lax.fori_loop(0, (nch - 1) // 2, _pair, 0)

        pltpu.make_async_copy(dst_hbm.at[pl.ds(base + (nch - 1) * _K, _K)],
                              di0, dm0).wait()
        pltpu.make_async_copy(u_hbm.at[sidx.at[pl.ds((nch - 1) * _K, _K)]],
                              rows0, sem0).wait()
        pltpu.sync_copy(rows0, acc.at[di0], add=True)

        plsc.subcore_barrier()
        pltpu.sync_copy(acc.at[pl.ds(s * rpt, rpt)],
                        out_hbm.at[c, pl.ds(s * rpt, rpt)])

    return agg_kernel(u, src, dst)


def _tc_pre(xp, w1, degp):
    nb = _NP // _RB

    def body(x_ref, w_ref, d0_ref, d1_ref, u_ref, dv_ref):
        deg = d0_ref[0, :, 0:1] + d1_ref[0, :, 0:1] + 1.0
        dv = lax.rsqrt(deg)
        dv_ref[...] = dv
        u_ref[...] = jnp.dot(x_ref[...], w_ref[...],
                             preferred_element_type=jnp.float32) * dv

    return pl.pallas_call(
        body,
        grid=(nb,),
        in_specs=[
            pl.BlockSpec((_RB, _D), lambda i: (i, 0)),
            pl.BlockSpec((_D, _D), lambda i: (0, 0)),
            pl.BlockSpec((1, _RB, _D), lambda i: (0, i, 0)),
            pl.BlockSpec((1, _RB, _D), lambda i: (1, i, 0)),
        ],
        out_specs=[
            pl.BlockSpec((_RB, _D), lambda i: (i, 0)),
            pl.BlockSpec((_RB, 1), lambda i: (i, 0)),
        ],
        out_shape=[
            jax.ShapeDtypeStruct((_NP, _D), jnp.float32),
            jax.ShapeDtypeStruct((_NP, 1), jnp.float32),
        ],
    )(xp, w1, degp, degp)


def _tc_mid(u1, p, dinv, b1, w2):
    nb = _NP // _RB

    def body(u_ref, p0_ref, p1_ref, dv_ref, b_ref, w_ref, o_ref):
        dv = dv_ref[...]
        s = (u_ref[...] + p0_ref[0] + p1_ref[0]) * dv + b_ref[...]
        h = jnp.maximum(s, 0.0)
        o_ref[...] = jnp.dot(h, w_ref[...],
                             preferred_element_type=jnp.float32) * dv

    return pl.pallas_call(
        body,
        grid=(nb,),
        in_specs=[
            pl.BlockSpec((_RB, _D), lambda i: (i, 0)),
            pl.BlockSpec((1, _RB, _D), lambda i: (0, i, 0)),
            pl.BlockSpec((1, _RB, _D), lambda i: (1, i, 0)),
            pl.BlockSpec((_RB, 1), lambda i: (i, 0)),
            pl.BlockSpec((1, _D), lambda i: (0, 0)),
            pl.BlockSpec((_D, _D), lambda i: (0, 0)),
        ],
        out_specs=pl.BlockSpec((_RB, _D), lambda i: (i, 0)),
        out_shape=jax.ShapeDtypeStruct((_NP, _D), jnp.float32),
    )(u1, p, p, dinv, b1, w2)


def _tc_post(u2, q, dinv, b2, batch2d, wlin, blin):
    nb = _NP // _RB
    c_out = wlin.shape[1]

    def body(u_ref, q0_ref, q1_ref, dv_ref, b_ref, bt_ref, wl_ref, bl_ref,
             o_ref, pool_acc, cnt_acc):
        i = pl.program_id(0)
        dv = dv_ref[...]
        s = (u_ref[...] + q0_ref[0] + q1_ref[0]) * dv + b_ref[...]
        h = jnp.maximum(s, 0.0)
        oh = (bt_ref[...] == lax.broadcasted_iota(jnp.int32, (_RB, _G), 1))
        oh = oh.astype(jnp.float32)
        pp = lax.dot_general(oh, h, (((0,), (0,)), ((), ())),
                             preferred_element_type=jnp.float32)
        cp = lax.dot_general(oh, jnp.ones((_RB, 1), jnp.float32),
                             (((0,), (0,)), ((), ())),
                             preferred_element_type=jnp.float32)

        @pl.when(i == 0)
        def _():
            pool_acc[...] = pp
            cnt_acc[...] = cp

        @pl.when(i > 0)
        def _():
            pool_acc[...] += pp
            cnt_acc[...] += cp

        @pl.when(i == nb - 1)
        def _():
            pooled = pool_acc[...] / jnp.maximum(cnt_acc[...], 1.0)
            o_ref[...] = jnp.dot(pooled, wl_ref[...],
                                 preferred_element_type=jnp.float32) + bl_ref[...]

    return pl.pallas_call(
        body,
        grid=(nb,),
        in_specs=[
            pl.BlockSpec((_RB, _D), lambda i: (i, 0)),
            pl.BlockSpec((1, _RB, _D), lambda i: (0, i, 0)),
            pl.BlockSpec((1, _RB, _D), lambda i: (1, i, 0)),
            pl.BlockSpec((_RB, 1), lambda i: (i, 0)),
            pl.BlockSpec((1, _D), lambda i: (0, 0)),
            pl.BlockSpec((_RB, 1), lambda i: (i, 0)),
            pl.BlockSpec((_D, c_out), lambda i: (0, 0)),
            pl.BlockSpec((1, c_out), lambda i: (0, 0)),
        ],
        out_specs=pl.BlockSpec((_G, c_out), lambda i: (0, 0)),
        out_shape=jax.ShapeDtypeStruct((_G, c_out), jnp.float32),
        scratch_shapes=[
            pltpu.VMEM((_G, _D), jnp.float32),
            pltpu.VMEM((_G, 1), jnp.float32),
        ],
    )(u2, q, q, dinv, b2, batch2d, wlin, blin)


def kernel(x, edge_index, batch, W1, b1, W2, b2, Wlin, blin):
    src = edge_index[0]
    dst = edge_index[1]
    xp = jnp.pad(x, ((0, _NP - _N), (0, 0)))
    batchp = jnp.pad(batch, (0, _NP - _N), constant_values=_G)
    degp = _sc_count(dst)
    u1, dinv = _tc_pre(xp, W1, degp)
    p = _sc_aggregate(u1, src, dst)
    u2 = _tc_mid(u1, p, dinv, b1.reshape(1, _D), W2)
    q = _sc_aggregate(u2, src, dst)
    return _tc_post(u2, q, dinv, b2.reshape(1, _D),
                    batchp.reshape(_NP, 1), Wlin, blin.reshape(1, -1))


# speedup vs baseline: 28.9903x; 1.1241x over previous
import functools

import jax
import jax.numpy as jnp
from jax import lax
from jax.experimental import pallas as pl
from jax.experimental.pallas import tpu as pltpu
from jax.experimental.pallas import tpu_sc as plsc

_N = 10000
_NP = 10240
_D = 128
_G = 64
_NC = 2
_NS = 16
_NW = _NC * _NS
_K = 80
_RB = 512


def _sc_count(dst):
    e = dst.shape[0]
    ew = e // _NW
    nch = ew // _K
    rpt = _NP // _NS
    mesh = plsc.VectorSubcoreMesh(core_axis_name="c", subcore_axis_name="s")

    @functools.partial(
        pl.kernel,
        out_type=jax.ShapeDtypeStruct((_NC, _NP, _D), jnp.float32),
        mesh=mesh,
        scratch_types=[
            pltpu.VMEM_SHARED((_NP, _D), jnp.float32),
            pltpu.VMEM((_K,), jnp.int32),
            pltpu.VMEM((_K,), jnp.int32),
            pltpu.VMEM((_K, _D), jnp.float32),
            pltpu.VMEM((128, _D), jnp.float32),
            pltpu.SemaphoreType.DMA,
            pltpu.SemaphoreType.DMA,
        ],
    )
    def cnt_kernel(dst_hbm, out_hbm, acc, di0, di1, ones_b, zb, dm0, dm1):
        c = lax.axis_index("c")
        s = lax.axis_index("s")
        wid = c * _NS + s
        base = wid * ew
        z16 = jnp.zeros((16,), jnp.float32)
        o16 = jnp.full((16,), 1.0, jnp.float32)

        def _fill(i, _):
            for j in range(_D // 16):
                zb[i, pl.ds(j * 16, 16)] = z16
            return 0

        lax.fori_loop(0, 128, _fill, 0)

        def _fill1(i, _):
            for j in range(_D // 16):
                ones_b[i, pl.ds(j * 16, 16)] = o16
            return 0

        lax.fori_loop(0, _K, _fill1, 0)

        def _zcp(j, _):
            pltpu.sync_copy(zb, acc.at[pl.ds(s * rpt + j * 128, 128)])
            return 0

        lax.fori_loop(0, rpt // 128, _zcp, 0)
        plsc.subcore_barrier()

        di = (di0, di1)
        dm = (dm0, dm1)

        def _issue(ch, b):
            pltpu.async_copy(dst_hbm.at[pl.ds(base + ch * _K, _K)],
                             di[b], dm[b])

        def _retire(ch, b):
            pltpu.make_async_copy(dst_hbm.at[pl.ds(base + ch * _K, _K)],
                                  di[b], dm[b]).wait()
            pltpu.sync_copy(ones_b, acc.at[di[b]], add=True)

        _issue(0, 0)

        def _pair(k, _):
            c0 = 2 * k
            _issue(c0 + 1, 1)
            _retire(c0, 0)
            _issue(c0 + 2, 0)
            _retire(c0 + 1, 1)
            return 0

        lax.fori_loop(0, (nch - 1) // 2, _pair, 0)
        _retire(nch - 1, 0)

        plsc.subcore_barrier()
        pltpu.sync_copy(acc.at[pl.ds(s * rpt, rpt)],
                        out_hbm.at[c, pl.ds(s * rpt, rpt)])

    return cnt_kernel(dst)


def _sc_aggregate(u, src, dst):
    e = src.shape[0]
    ew = e // _NW
    nch = ew // _K
    assert nch >= 2 and (nch - 2) % 3 == 0
    rpt = _NP // _NS
    mesh = plsc.VectorSubcoreMesh(core_axis_name="c", subcore_axis_name="s")

    @functools.partial(
        pl.kernel,
        out_type=jax.ShapeDtypeStruct((_NC, _NP, _D), jnp.float32),
        mesh=mesh,
        scratch_types=[
            pltpu.VMEM_SHARED((_NP, _D), jnp.float32),
            pltpu.VMEM((ew,), jnp.int32),
            pltpu.VMEM((_K,), jnp.int32),
            pltpu.VMEM((_K,), jnp.int32),
            pltpu.VMEM((_K,), jnp.int32),
            pltpu.VMEM((_K, _D), jnp.float32),
            pltpu.VMEM((_K, _D), jnp.float32),
            pltpu.VMEM((_K, _D), jnp.float32),
            pltpu.VMEM((16, _D), jnp.float32),
            pltpu.SemaphoreType.DMA,
            pltpu.SemaphoreType.DMA,
            pltpu.SemaphoreType.DMA,
            pltpu.SemaphoreType.DMA,
            pltpu.SemaphoreType.DMA,
            pltpu.SemaphoreType.DMA,
        ],
    )
    def agg_kernel(u_hbm, src_hbm, dst_hbm, out_hbm, acc, sidx,
                   di0, di1, di2, rows0, rows1, rows2, zb,
                   sem0, sem1, sem2, dm0, dm1, dm2):
        c = lax.axis_index("c")
        s = lax.axis_index("s")
        wid = c * _NS + s
        base = wid * ew
        z16 = jnp.zeros((16,), jnp.float32)

        def _fill(i, _):
            for j in range(_D // 16):
                zb[i, pl.ds(j * 16, 16)] = z16
            return 0

        lax.fori_loop(0, 16, _fill, 0)

        pltpu.sync_copy(src_hbm.at[pl.ds(base, ew)], sidx)

        def _zcp(j, _):
            pltpu.sync_copy(zb, acc.at[pl.ds(s * rpt + j * 16, 16)])
            return 0

        lax.fori_loop(0, rpt // 16, _zcp, 0)
        plsc.subcore_barrier()

        di = (di0, di1, di2)
        dm = (dm0, dm1, dm2)
        rows = (rows0, rows1, rows2)
        sem = (sem0, sem1, sem2)

        def _issue(ch, b):
            pltpu.async_copy(dst_hbm.at[pl.ds(base + ch * _K, _K)],
                             di[b], dm[b])
            pltpu.async_copy(u_hbm.at[sidx.at[pl.ds(ch * _K, _K)]],
                             rows[b], sem[b])

        def _retire(ch, b):
            pltpu.make_async_copy(dst_hbm.at[pl.ds(base + ch * _K, _K)],
                                  di[b], dm[b]).wait()
            pltpu.make_async_copy(u_hbm.at[sidx.at[pl.ds(ch * _K, _K)]],
                                  rows[b], sem[b]).wait()
            pltpu.sync_copy(rows[b], acc.at[di[b]], add=True)

        _issue(0, 0)
        _issue(1, 1)

        def _trip(k, _):
            c0 = 3 * k
            for j in range(3):
                _issue(c0 + j + 2, (j + 2) % 3)
                _retire(c0 + j, j)
            return 0

        lax.fori_loop(0, (nch - 2) // 3, _trip, 0)

        _retire(nch - 2, (nch - 2) % 3)
        _retire(nch - 1, (nch - 1) % 3)

        plsc.subcore_barrier()
        pltpu.sync_copy(acc.at[pl.ds(s * rpt, rpt)],
                        out_hbm.at[c, pl.ds(s * rpt, rpt)])

    return agg_kernel(u, src, dst)


def _tc_pre(xp, w1, degp):
    nb = _NP // _RB

    def body(x_ref, w_ref, d0_ref, d1_ref, u_ref, dv_ref):
        deg = d0_ref[0, :, 0:1] + d1_ref[0, :, 0:1] + 1.0
        dv = lax.rsqrt(deg)
        dv_ref[...] = dv
        u_ref[...] = jnp.dot(x_ref[...], w_ref[...],
                             preferred_element_type=jnp.float32) * dv

    return pl.pallas_call(
        body,
        grid=(nb,),
        in_specs=[
            pl.BlockSpec((_RB, _D), lambda i: (i, 0)),
            pl.BlockSpec((_D, _D), lambda i: (0, 0)),
            pl.BlockSpec((1, _RB, _D), lambda i: (0, i, 0)),
            pl.BlockSpec((1, _RB, _D), lambda i: (1, i, 0)),
        ],
        out_specs=[
            pl.BlockSpec((_RB, _D), lambda i: (i, 0)),
            pl.BlockSpec((_RB, 1), lambda i: (i, 0)),
        ],
        out_shape=[
            jax.ShapeDtypeStruct((_NP, _D), jnp.float32),
            jax.ShapeDtypeStruct((_NP, 1), jnp.float32),
        ],
    )(xp, w1, degp, degp)


def _tc_mid(u1, p, dinv, b1, w2):
    nb = _NP // _RB

    def body(u_ref, p0_ref, p1_ref, dv_ref, b_ref, w_ref, o_ref):
        dv = dv_ref[...]
        s = (u_ref[...] + p0_ref[0] + p1_ref[0]) * dv + b_ref[...]
        h = jnp.maximum(s, 0.0)
        o_ref[...] = jnp.dot(h, w_ref[...],
                             preferred_element_type=jnp.float32) * dv

    return pl.pallas_call(
        body,
        grid=(nb,),
        in_specs=[
            pl.BlockSpec((_RB, _D), lambda i: (i, 0)),
            pl.BlockSpec((1, _RB, _D), lambda i: (0, i, 0)),
            pl.BlockSpec((1, _RB, _D), lambda i: (1, i, 0)),
            pl.BlockSpec((_RB, 1), lambda i: (i, 0)),
            pl.BlockSpec((1, _D), lambda i: (0, 0)),
            pl.BlockSpec((_D, _D), lambda i: (0, 0)),
        ],
        out_specs=pl.BlockSpec((_RB, _D), lambda i: (i, 0)),
        out_shape=jax.ShapeDtypeStruct((_NP, _D), jnp.float32),
    )(u1, p, p, dinv, b1, w2)


def _tc_post(u2, q, dinv, b2, batch2d, wlin, blin):
    nb = _NP // _RB
    c_out = wlin.shape[1]

    def body(u_ref, q0_ref, q1_ref, dv_ref, b_ref, bt_ref, wl_ref, bl_ref,
             o_ref, pool_acc, cnt_acc):
        i = pl.program_id(0)
        dv = dv_ref[...]
        s = (u_ref[...] + q0_ref[0] + q1_ref[0]) * dv + b_ref[...]
        h = jnp.maximum(s, 0.0)
        oh = (bt_ref[...] == lax.broadcasted_iota(jnp.int32, (_RB, _G), 1))
        oh = oh.astype(jnp.float32)
        pp = lax.dot_general(oh, h, (((0,), (0,)), ((), ())),
                             preferred_element_type=jnp.float32)
        cp = lax.dot_general(oh, jnp.ones((_RB, 1), jnp.float32),
                             (((0,), (0,)), ((), ())),
                             preferred_element_type=jnp.float32)

        @pl.when(i == 0)
        def _():
            pool_acc[...] = pp
            cnt_acc[...] = cp

        @pl.when(i > 0)
        def _():
            pool_acc[...] += pp
            cnt_acc[...] += cp

        @pl.when(i == nb - 1)
        def _():
            pooled = pool_acc[...] / jnp.maximum(cnt_acc[...], 1.0)
            o_ref[...] = jnp.dot(pooled, wl_ref[...],
                                 preferred_element_type=jnp.float32) + bl_ref[...]

    return pl.pallas_call(
        body,
        grid=(nb,),
        in_specs=[
            pl.BlockSpec((_RB, _D), lambda i: (i, 0)),
            pl.BlockSpec((1, _RB, _D), lambda i: (0, i, 0)),
            pl.BlockSpec((1, _RB, _D), lambda i: (1, i, 0)),
            pl.BlockSpec((_RB, 1), lambda i: (i, 0)),
            pl.BlockSpec((1, _D), lambda i: (0, 0)),
            pl.BlockSpec((_RB, 1), lambda i: (i, 0)),
            pl.BlockSpec((_D, c_out), lambda i: (0, 0)),
            pl.BlockSpec((1, c_out), lambda i: (0, 0)),
        ],
        out_specs=pl.BlockSpec((_G, c_out), lambda i: (0, 0)),
        out_shape=jax.ShapeDtypeStruct((_G, c_out), jnp.float32),
        scratch_shapes=[
            pltpu.VMEM((_G, _D), jnp.float32),
            pltpu.VMEM((_G, 1), jnp.float32),
        ],
    )(u2, q, q, dinv, b2, batch2d, wlin, blin)


def kernel(x, edge_index, batch, W1, b1, W2, b2, Wlin, blin):
    src = edge_index[0]
    dst = edge_index[1]
    xp = jnp.pad(x, ((0, _NP - _N), (0, 0)))
    batchp = jnp.pad(batch, (0, _NP - _N), constant_values=_G)
    degp = _sc_count(dst)
    u1, dinv = _tc_pre(xp, W1, degp)
    p = _sc_aggregate(u1, src, dst)
    u2 = _tc_mid(u1, p, dinv, b1.reshape(1, _D), W2)
    q = _sc_aggregate(u2, src, dst)
    return _tc_post(u2, q, dinv, b2.reshape(1, _D),
                    batchp.reshape(_NP, 1), Wlin, blin.reshape(1, -1))
